# baseline (speedup 1.0000x reference)
import functools
import math

import numpy as np

import jax
import jax.numpy as jnp
from jax.experimental import pallas as pl
from jax.experimental.pallas import tpu as pltpu

_PADL = 8
_PADW = 16


def _interp_matrix_np(out_size, in_size):
    scale = in_size / out_size
    i = np.arange(out_size, dtype=np.float64)
    src = np.maximum((i + 0.5) * scale - 0.5, 0.0)
    i0 = np.minimum(np.floor(src).astype(np.int64), in_size - 1)
    i1 = np.minimum(i0 + 1, in_size - 1)
    frac = src - i0
    m = np.zeros((out_size, in_size), dtype=np.float64)
    m[np.arange(out_size), i0] += 1.0 - frac
    m[np.arange(out_size), i1] += frac
    return m.astype(np.float32)


@functools.lru_cache(maxsize=None)
def _upsample_kron_np(hout, wout, hin, win):
    a = _interp_matrix_np(hout, hin)
    b = _interp_matrix_np(wout, win)
    k = np.einsum('ia,jb->ijab', a, b).reshape(hout * wout, hin * win)
    return k.astype(np.float32)


def _row_slab(xp_ref, cin, H, W):
    hp = H + 2
    return jnp.concatenate(
        [xp_ref[:, _PADL - 1 + dx:_PADL - 1 + dx + W, :cin].reshape(hp * W, cin)
         for dx in range(3)], axis=1)


def _row_dots(col3, w_ref, b_ref, cin, H, W):
    y = [jnp.dot(col3, w_ref[3 * cin * dy:3 * cin * (dy + 1), :],
                 preferred_element_type=jnp.float32) for dy in range(3)]
    out = y[0][0:H * W] + y[1][W:W + H * W] + y[2][2 * W:2 * W + H * W]
    return out + b_ref[...]


def _conv(x2d, pad_ref, w_ref, b_ref, H, W, relu_in):
    cin = w_ref.shape[0] // 9
    xin = jnp.maximum(x2d, 0.0) if relu_in else x2d
    pad_ref[1:H + 1, _PADL:_PADL + W, :cin] = \
        xin.reshape(H, W, cin).astype(jnp.bfloat16)
    col3 = _row_slab(pad_ref, cin, H, W)
    return _row_dots(col3, w_ref, b_ref, cin, H, W)


def _conv_from_input(f_ref, w_ref, b_ref, H, W):
    cin = w_ref.shape[0] // 9
    col3 = _row_slab(f_ref, cin, H, W)
    return _row_dots(col3, w_ref, b_ref, cin, H, W)


def _pred_diff(m, wd_ref, bd_ref, H, W):
    hw = H * W
    q = jnp.dot(jnp.maximum(m, 0.0).astype(jnp.bfloat16), wd_ref[...],
                preferred_element_type=jnp.float32)
    qp = jnp.pad(q, ((W + 1, W + 1), (0, 0)))
    wcol = jax.lax.broadcasted_iota(jnp.int32, (hw, 1), 0) % W
    d = jnp.zeros((hw, 1), jnp.float32)
    for dy in range(3):
        for dx in range(3):
            k = dy * 3 + dx
            s = (dy - 1) * W + (dx - 1)
            col = qp[W + 1 + s:W + 1 + s + hw, k:k + 1]
            if dx == 0:
                col = jnp.where(wcol == 0, 0.0, col)
            elif dx == 2:
                col = jnp.where(wcol == W - 1, 0.0, col)
            d = d + col
    return d + bd_ref[...]


def _zero_border(pad_ref, H, W):
    c = pad_ref.shape[-1]
    wp = pad_ref.shape[1]
    z_row = jnp.zeros((1, wp, c), jnp.bfloat16)
    z_col = jnp.zeros((H + 2, 1, c), jnp.bfloat16)
    pad_ref[0:1, :, :] = z_row
    pad_ref[H + 1:H + 2, :, :] = z_row
    pad_ref[:, _PADL - 1:_PADL, :] = z_col
    pad_ref[:, _PADL + W:_PADL + W + 1, :] = z_col


def _pipe_kernel(f3_ref, f2_ref, f1_ref, kr2_ref, kr1_ref,
                 w0, b0, w1, b1, w2, b2,
                 fs2, bfs2, a2, ba2, c2, bc2, d2, bd2, e2, be2,
                 fs1, bfs1, a1, ba1, c1, bc1, d1, bd1, e1, be1,
                 wd, bdd, out_ref, pad3, pad2, pad1,
                 *, H3, W3, H2, W2, H1, W1):
    _zero_border(pad3, H3, W3)
    _zero_border(pad2, H2, W2)
    _zero_border(pad1, H1, W1)

    x = _conv_from_input(f3_ref, w0, b0, H3, W3)
    r = _conv(x, pad3, w1, b1, H3, W3, relu_in=True)
    r = _conv(r, pad3, w2, b2, H3, W3, relu_in=True)
    m3 = (x + r).astype(jnp.bfloat16)

    up = jnp.dot(kr2_ref[...], m3, preferred_element_type=jnp.float32)
    x = _conv_from_input(f2_ref, fs2, bfs2, H2, W2)
    r = _conv(x, pad2, a2, ba2, H2, W2, relu_in=True)
    r = _conv(r, pad2, c2, bc2, H2, W2, relu_in=True)
    m = x + r + up
    r = _conv(m, pad2, d2, bd2, H2, W2, relu_in=True)
    m2 = (m + _conv(r, pad2, e2, be2, H2, W2, relu_in=True)).astype(jnp.bfloat16)

    up = jnp.dot(kr1_ref[...], m2, preferred_element_type=jnp.float32)
    x = _conv_from_input(f1_ref, fs1, bfs1, H1, W1)
    r = _conv(x, pad1, a1, ba1, H1, W1, relu_in=True)
    r = _conv(r, pad1, c1, bc1, H1, W1, relu_in=True)
    m = x + r + up
    r = _conv(m, pad1, d1, bd1, H1, W1, relu_in=True)
    m = m + _conv(r, pad1, e1, be1, H1, W1, relu_in=True)
    d = _pred_diff(m, wd, bdd, H1, W1)
    out_ref[...] = d.reshape(1, H1 * W1)


def _pipeline(f3, f2, f1, kr2, kr1, weights, dims):
    H3, W3, H2, W2, H1, W1 = dims
    N = f3.shape[0]
    args = [f3, f2, f1, kr2, kr1] + weights
    in_specs = [pl.BlockSpec((None,) + f3.shape[1:], lambda n: (n, 0, 0, 0)),
                pl.BlockSpec((None,) + f2.shape[1:], lambda n: (n, 0, 0, 0)),
                pl.BlockSpec((None,) + f1.shape[1:], lambda n: (n, 0, 0, 0))]
    in_specs += [pl.BlockSpec(a.shape, lambda n: (0, 0)) for a in args[3:]]
    kern = functools.partial(_pipe_kernel, H3=H3, W3=W3, H2=H2, W2=W2,
                             H1=H1, W1=W1)
    return pl.pallas_call(
        kern,
        out_shape=jax.ShapeDtypeStruct((N, 1, H1 * W1), jnp.float32),
        grid=(N,),
        in_specs=in_specs,
        out_specs=pl.BlockSpec((None, 1, H1 * W1), lambda n: (n, 0, 0)),
        scratch_shapes=[
            pltpu.VMEM((H3 + 2, W3 + _PADW, 128), jnp.bfloat16),
            pltpu.VMEM((H2 + 2, W2 + _PADW, 128), jnp.bfloat16),
            pltpu.VMEM((H1 + 2, W1 + _PADW, 128), jnp.bfloat16),
        ],
        compiler_params=pltpu.CompilerParams(dimension_semantics=("arbitrary",)),
    )(*args)


def _agg_kernel(d_ref, a4_ref, bt_ref, out_ref, *, no, max_obj, hf, wf):
    eps = 1e-7

    def logit(e):
        e = jnp.clip(e, eps, 1.0 - eps)
        return jnp.log(e / (1.0 - e))

    t = jnp.dot(a4_ref[...], d_ref[...], preferred_element_type=jnp.float32)
    u = jnp.dot(t, bt_ref[...], preferred_element_type=jnp.float32)
    ps = jax.nn.sigmoid(u)
    rows = [ps[i * hf:(i + 1) * hf] for i in range(no)]
    bg = 1.0 - rows[0]
    for i in range(1, no):
        bg = bg * (1.0 - rows[i])
    out_ref[0:1] = logit(bg)[None]
    for i in range(no):
        out_ref[1 + i:2 + i] = logit(rows[i])[None]
    pad_val = math.log(eps / (1.0 - eps))
    for j in range(no + 1, max_obj):
        out_ref[j:j + 1] = jnp.full((1, hf, wf), pad_val, jnp.float32)


def _aggregate(d3, a4, bt, no, max_obj, hf, wf):
    bs = d3.shape[0]
    args = [d3, a4, bt]
    in_specs = [pl.BlockSpec((None,) + d3.shape[1:], lambda n: (n, 0, 0)),
                pl.BlockSpec(a4.shape, lambda n: (0, 0)),
                pl.BlockSpec(bt.shape, lambda n: (0, 0))]
    return pl.pallas_call(
        functools.partial(_agg_kernel, no=no, max_obj=max_obj, hf=hf, wf=wf),
        out_shape=jax.ShapeDtypeStruct((bs, max_obj, hf, wf), jnp.float32),
        grid=(bs,),
        in_specs=in_specs,
        out_specs=pl.BlockSpec((None, max_obj, hf, wf), lambda n: (n, 0, 0, 0)),
        compiler_params=pltpu.CompilerParams(dimension_semantics=("arbitrary",)),
    )(*args)


def _to_padded_windows(x):
    N, C, H, W = x.shape
    t = jnp.transpose(x.reshape(N, C, H * W), (0, 2, 1)).astype(jnp.bfloat16)
    t = t.reshape(N, H, W, C)
    return jnp.pad(t, ((0, 0), (1, 1), (_PADL, _PADW - _PADL), (0, 0)))


def kernel(convFM_w, convFM_b, resMM_w1, resMM_b1, resMM_w2, resMM_b2,
           rf2_convFS_w, rf2_convFS_b, rf2_resFS_w1, rf2_resFS_b1,
           rf2_resFS_w2, rf2_resFS_b2, rf2_resMM_w1, rf2_resMM_b1,
           rf2_resMM_w2, rf2_resMM_b2, rf1_convFS_w, rf1_convFS_b,
           rf1_resFS_w1, rf1_resFS_b1, rf1_resFS_w2, rf1_resFS_b2,
           rf1_resMM_w1, rf1_resMM_b1, rf1_resMM_w2, rf1_resMM_b2,
           pred_w, pred_b, opt_feat, r2, r1):
    bs, max_obj, h, w = 4, 5, 256, 256
    N, c_in, H3, W3 = opt_feat.shape
    _, c_in_2, H2, W2 = r2.shape
    _, c_in_1, H1, W1 = r1.shape
    no = N // bs

    bf16 = jnp.bfloat16
    cast = lambda a: a.astype(bf16)

    f3 = _to_padded_windows(opt_feat)
    f2 = _to_padded_windows(r2)
    f1 = _to_padded_windows(r1)

    kr2 = jnp.asarray(_upsample_kron_np(H2, W2, H3, W3), bf16)
    kr1 = jnp.asarray(_upsample_kron_np(H1, W1, H2, W2), bf16)

    wd = cast((pred_w[:, 1] - pred_w[:, 0]).reshape(9, pred_w.shape[0] // 9).T)
    bd = (pred_b[:, 1:2] - pred_b[:, 0:1]).astype(jnp.float32)

    weights = [kr2, kr1,
               cast(convFM_w), convFM_b, cast(resMM_w1), resMM_b1,
               cast(resMM_w2), resMM_b2,
               cast(rf2_convFS_w), rf2_convFS_b,
               cast(rf2_resFS_w1), rf2_resFS_b1,
               cast(rf2_resFS_w2), rf2_resFS_b2,
               cast(rf2_resMM_w1), rf2_resMM_b1,
               cast(rf2_resMM_w2), rf2_resMM_b2,
               cast(rf1_convFS_w), rf1_convFS_b,
               cast(rf1_resFS_w1), rf1_resFS_b1,
               cast(rf1_resFS_w2), rf1_resFS_b2,
               cast(rf1_resMM_w1), rf1_resMM_b1,
               cast(rf1_resMM_w2), rf1_resMM_b2,
               wd, bd]

    d = _pipeline(f3, f2, f1, weights[0], weights[1], weights[2:],
                  (H3, W3, H2, W2, H1, W1))

    d3 = d.reshape(bs, no * H1, W1)
    a4 = jnp.asarray(np.kron(np.eye(no, dtype=np.float32),
                             _interp_matrix_np(h, H1)), jnp.float32)
    bt = jnp.asarray(_interp_matrix_np(w, W1).T, jnp.float32)
    return _aggregate(d3, a4, bt, no, max_obj, h, w)


# speedup vs baseline: 1.0134x; 1.0134x over previous
import functools
import math

import numpy as np

import jax
import jax.numpy as jnp
from jax.experimental import pallas as pl
from jax.experimental.pallas import tpu as pltpu

_PADL = 8
_PADW = 16


def _interp_matrix_np(out_size, in_size):
    scale = in_size / out_size
    i = np.arange(out_size, dtype=np.float64)
    src = np.maximum((i + 0.5) * scale - 0.5, 0.0)
    i0 = np.minimum(np.floor(src).astype(np.int64), in_size - 1)
    i1 = np.minimum(i0 + 1, in_size - 1)
    frac = src - i0
    m = np.zeros((out_size, in_size), dtype=np.float64)
    m[np.arange(out_size), i0] += 1.0 - frac
    m[np.arange(out_size), i1] += frac
    return m.astype(np.float32)


@functools.lru_cache(maxsize=None)
def _upsample_kron_np(hout, wout, hin, win):
    a = _interp_matrix_np(hout, hin)
    b = _interp_matrix_np(wout, win)
    k = np.einsum('ia,jb->ijab', a, b).reshape(hout * wout, hin * win)
    return k.astype(np.float32)


def _row_slab(xp_ref, cin, H, W):
    hp = H + 2
    return jnp.concatenate(
        [xp_ref[:, _PADL - 1 + dx:_PADL - 1 + dx + W, :cin].reshape(hp * W, cin)
         for dx in range(3)], axis=1)


def _row_dots(col3, w_ref, b_ref, cin, H, W):
    y = [jnp.dot(col3, w_ref[3 * cin * dy:3 * cin * (dy + 1), :],
                 preferred_element_type=jnp.float32) for dy in range(3)]
    out = y[0][0:H * W] + y[1][W:W + H * W] + y[2][2 * W:2 * W + H * W]
    return out + b_ref[...]


def _conv(x2d, pad_ref, w_ref, b_ref, H, W, relu_in):
    cin = w_ref.shape[0] // 9
    xin = jnp.maximum(x2d, 0.0) if relu_in else x2d
    pad_ref[1:H + 1, _PADL:_PADL + W, :cin] = \
        xin.reshape(H, W, cin).astype(jnp.bfloat16)
    col3 = _row_slab(pad_ref, cin, H, W)
    return _row_dots(col3, w_ref, b_ref, cin, H, W)


def _conv_from_input(f_ref, w_ref, b_ref, H, W):
    cin = w_ref.shape[0] // 9
    col3 = _row_slab(f_ref, cin, H, W)
    return _row_dots(col3, w_ref, b_ref, cin, H, W)


def _pred_diff(m, wd_ref, bd_ref, H, W):
    hw = H * W
    q = jnp.dot(jnp.maximum(m, 0.0).astype(jnp.bfloat16), wd_ref[...],
                preferred_element_type=jnp.float32)
    qp = jnp.pad(q, ((W + 1, W + 1), (0, 0)))
    wcol = jax.lax.broadcasted_iota(jnp.int32, (hw, 1), 0) % W
    d = jnp.zeros((hw, 1), jnp.float32)
    for dy in range(3):
        for dx in range(3):
            k = dy * 3 + dx
            s = (dy - 1) * W + (dx - 1)
            col = qp[W + 1 + s:W + 1 + s + hw, k:k + 1]
            if dx == 0:
                col = jnp.where(wcol == 0, 0.0, col)
            elif dx == 2:
                col = jnp.where(wcol == W - 1, 0.0, col)
            d = d + col
    return d + bd_ref[...]


def _zero_border(pad_ref, H, W):
    c = pad_ref.shape[-1]
    wp = pad_ref.shape[1]
    z_row = jnp.zeros((1, wp, c), jnp.bfloat16)
    z_col = jnp.zeros((H + 2, 1, c), jnp.bfloat16)
    pad_ref[0:1, :, :] = z_row
    pad_ref[H + 1:H + 2, :, :] = z_row
    pad_ref[:, _PADL - 1:_PADL, :] = z_col
    pad_ref[:, _PADL + W:_PADL + W + 1, :] = z_col


def _pipe_kernel(f3_ref, f2_ref, f1_ref, kr2_ref, kr1_ref,
                 w0, b0, w1, b1, w2, b2,
                 fs2, bfs2, a2, ba2, c2, bc2, d2, bd2, e2, be2,
                 fs1, bfs1, a1, ba1, c1, bc1, d1, bd1, e1, be1,
                 wd, bdd, out_ref, pad3, pad2, pad1,
                 *, H3, W3, H2, W2, H1, W1):
    _zero_border(pad3, H3, W3)
    _zero_border(pad2, H2, W2)
    _zero_border(pad1, H1, W1)

    x = _conv_from_input(f3_ref, w0, b0, H3, W3)
    r = _conv(x, pad3, w1, b1, H3, W3, relu_in=True)
    r = _conv(r, pad3, w2, b2, H3, W3, relu_in=True)
    m3 = (x + r).astype(jnp.bfloat16)

    up = jnp.dot(kr2_ref[...], m3, preferred_element_type=jnp.float32)
    x = _conv_from_input(f2_ref, fs2, bfs2, H2, W2)
    r = _conv(x, pad2, a2, ba2, H2, W2, relu_in=True)
    r = _conv(r, pad2, c2, bc2, H2, W2, relu_in=True)
    m = x + r + up
    r = _conv(m, pad2, d2, bd2, H2, W2, relu_in=True)
    m2 = (m + _conv(r, pad2, e2, be2, H2, W2, relu_in=True)).astype(jnp.bfloat16)

    up = jnp.dot(kr1_ref[...], m2, preferred_element_type=jnp.float32)
    x = _conv_from_input(f1_ref, fs1, bfs1, H1, W1)
    r = _conv(x, pad1, a1, ba1, H1, W1, relu_in=True)
    r = _conv(r, pad1, c1, bc1, H1, W1, relu_in=True)
    m = x + r + up
    r = _conv(m, pad1, d1, bd1, H1, W1, relu_in=True)
    m = m + _conv(r, pad1, e1, be1, H1, W1, relu_in=True)
    d = _conv(m, pad1, wd, bdd, H1, W1, relu_in=True)
    out_ref[...] = d.reshape(1, H1 * W1)


def _pipeline(f3, f2, f1, kr2, kr1, weights, dims):
    H3, W3, H2, W2, H1, W1 = dims
    N = f3.shape[0]
    args = [f3, f2, f1, kr2, kr1] + weights
    in_specs = [pl.BlockSpec((None,) + f3.shape[1:], lambda n: (n, 0, 0, 0)),
                pl.BlockSpec((None,) + f2.shape[1:], lambda n: (n, 0, 0, 0)),
                pl.BlockSpec((None,) + f1.shape[1:], lambda n: (n, 0, 0, 0))]
    in_specs += [pl.BlockSpec(a.shape, lambda n: (0, 0)) for a in args[3:]]
    kern = functools.partial(_pipe_kernel, H3=H3, W3=W3, H2=H2, W2=W2,
                             H1=H1, W1=W1)
    return pl.pallas_call(
        kern,
        out_shape=jax.ShapeDtypeStruct((N, 1, H1 * W1), jnp.float32),
        grid=(N,),
        in_specs=in_specs,
        out_specs=pl.BlockSpec((None, 1, H1 * W1), lambda n: (n, 0, 0)),
        scratch_shapes=[
            pltpu.VMEM((H3 + 2, W3 + _PADW, 128), jnp.bfloat16),
            pltpu.VMEM((H2 + 2, W2 + _PADW, 128), jnp.bfloat16),
            pltpu.VMEM((H1 + 2, W1 + _PADW, 128), jnp.bfloat16),
        ],
        compiler_params=pltpu.CompilerParams(dimension_semantics=("arbitrary",)),
    )(*args)


def _agg_kernel(d_ref, a4_ref, bt_ref, out_ref, *, no, max_obj, hf, wf):
    eps = 1e-7

    def logit(e):
        e = jnp.clip(e, eps, 1.0 - eps)
        return jnp.log(e / (1.0 - e))

    t = jnp.dot(a4_ref[...], d_ref[...], preferred_element_type=jnp.float32)
    u = jnp.dot(t, bt_ref[...], preferred_element_type=jnp.float32)
    ps = jax.nn.sigmoid(u)
    rows = [ps[i * hf:(i + 1) * hf] for i in range(no)]
    bg = 1.0 - rows[0]
    for i in range(1, no):
        bg = bg * (1.0 - rows[i])
    out_ref[0:1] = logit(bg)[None]
    for i in range(no):
        out_ref[1 + i:2 + i] = logit(rows[i])[None]
    pad_val = math.log(eps / (1.0 - eps))
    for j in range(no + 1, max_obj):
        out_ref[j:j + 1] = jnp.full((1, hf, wf), pad_val, jnp.float32)


def _aggregate(d3, a4, bt, no, max_obj, hf, wf):
    bs = d3.shape[0]
    args = [d3, a4, bt]
    in_specs = [pl.BlockSpec((None,) + d3.shape[1:], lambda n: (n, 0, 0)),
                pl.BlockSpec(a4.shape, lambda n: (0, 0)),
                pl.BlockSpec(bt.shape, lambda n: (0, 0))]
    return pl.pallas_call(
        functools.partial(_agg_kernel, no=no, max_obj=max_obj, hf=hf, wf=wf),
        out_shape=jax.ShapeDtypeStruct((bs, max_obj, hf, wf), jnp.float32),
        grid=(bs,),
        in_specs=in_specs,
        out_specs=pl.BlockSpec((None, max_obj, hf, wf), lambda n: (n, 0, 0, 0)),
        compiler_params=pltpu.CompilerParams(dimension_semantics=("arbitrary",)),
    )(*args)


def _to_padded_windows(x):
    N, C, H, W = x.shape
    t = jnp.transpose(x.reshape(N, C, H * W), (0, 2, 1)).astype(jnp.bfloat16)
    t = t.reshape(N, H, W, C)
    return jnp.pad(t, ((0, 0), (1, 1), (_PADL, _PADW - _PADL), (0, 0)))


def kernel(convFM_w, convFM_b, resMM_w1, resMM_b1, resMM_w2, resMM_b2,
           rf2_convFS_w, rf2_convFS_b, rf2_resFS_w1, rf2_resFS_b1,
           rf2_resFS_w2, rf2_resFS_b2, rf2_resMM_w1, rf2_resMM_b1,
           rf2_resMM_w2, rf2_resMM_b2, rf1_convFS_w, rf1_convFS_b,
           rf1_resFS_w1, rf1_resFS_b1, rf1_resFS_w2, rf1_resFS_b2,
           rf1_resMM_w1, rf1_resMM_b1, rf1_resMM_w2, rf1_resMM_b2,
           pred_w, pred_b, opt_feat, r2, r1):
    bs, max_obj, h, w = 4, 5, 256, 256
    N, c_in, H3, W3 = opt_feat.shape
    _, c_in_2, H2, W2 = r2.shape
    _, c_in_1, H1, W1 = r1.shape
    no = N // bs

    bf16 = jnp.bfloat16
    cast = lambda a: a.astype(bf16)

    f3 = _to_padded_windows(opt_feat)
    f2 = _to_padded_windows(r2)
    f1 = _to_padded_windows(r1)

    kr2 = jnp.asarray(_upsample_kron_np(H2, W2, H3, W3), bf16)
    kr1 = jnp.asarray(_upsample_kron_np(H1, W1, H2, W2), bf16)

    wd = cast(pred_w[:, 1:2] - pred_w[:, 0:1])
    bd = (pred_b[:, 1:2] - pred_b[:, 0:1]).astype(jnp.float32)

    weights = [kr2, kr1,
               cast(convFM_w), convFM_b, cast(resMM_w1), resMM_b1,
               cast(resMM_w2), resMM_b2,
               cast(rf2_convFS_w), rf2_convFS_b,
               cast(rf2_resFS_w1), rf2_resFS_b1,
               cast(rf2_resFS_w2), rf2_resFS_b2,
               cast(rf2_resMM_w1), rf2_resMM_b1,
               cast(rf2_resMM_w2), rf2_resMM_b2,
               cast(rf1_convFS_w), rf1_convFS_b,
               cast(rf1_resFS_w1), rf1_resFS_b1,
               cast(rf1_resFS_w2), rf1_resFS_b2,
               cast(rf1_resMM_w1), rf1_resMM_b1,
               cast(rf1_resMM_w2), rf1_resMM_b2,
               wd, bd]

    d = _pipeline(f3, f2, f1, weights[0], weights[1], weights[2:],
                  (H3, W3, H2, W2, H1, W1))

    d3 = d.reshape(bs, no * H1, W1)
    a4 = jnp.asarray(np.kron(np.eye(no, dtype=np.float32),
                             _interp_matrix_np(h, H1)), jnp.float32)
    bt = jnp.asarray(_interp_matrix_np(w, W1).T, jnp.float32)
    return _aggregate(d3, a4, bt, no, max_obj, h, w)


# speedup vs baseline: 1.0353x; 1.0216x over previous
import functools
import math

import numpy as np

import jax
import jax.numpy as jnp
from jax.experimental import pallas as pl
from jax.experimental.pallas import tpu as pltpu

_PADL = 8
_PADW = 16


def _interp_matrix_np(out_size, in_size):
    scale = in_size / out_size
    i = np.arange(out_size, dtype=np.float64)
    src = np.maximum((i + 0.5) * scale - 0.5, 0.0)
    i0 = np.minimum(np.floor(src).astype(np.int64), in_size - 1)
    i1 = np.minimum(i0 + 1, in_size - 1)
    frac = src - i0
    m = np.zeros((out_size, in_size), dtype=np.float64)
    m[np.arange(out_size), i0] += 1.0 - frac
    m[np.arange(out_size), i1] += frac
    return m.astype(np.float32)


@functools.lru_cache(maxsize=None)
def _upsample_kron_np(hout, wout, hin, win):
    a = _interp_matrix_np(hout, hin)
    b = _interp_matrix_np(wout, win)
    k = np.einsum('ia,jb->ijab', a, b).reshape(hout * wout, hin * win)
    return k.astype(np.float32)


def _row_slab(xp_ref, cin, H, W):
    hp = H + 2
    return jnp.concatenate(
        [xp_ref[:, _PADL - 1 + dx:_PADL - 1 + dx + W, :cin].reshape(hp * W, cin)
         for dx in range(3)], axis=1)


def _row_dots(col3, w_ref, b_ref, cin, H, W):
    y = [jnp.dot(col3[dy * W:dy * W + H * W],
                 w_ref[3 * cin * dy:3 * cin * (dy + 1), :],
                 preferred_element_type=jnp.float32) for dy in range(3)]
    return y[0] + y[1] + y[2] + b_ref[...]


def _conv(x2d, pad_ref, w_ref, b_ref, H, W, relu_in):
    cin = w_ref.shape[0] // 9
    xin = jnp.maximum(x2d, 0.0) if relu_in else x2d
    pad_ref[1:H + 1, _PADL:_PADL + W, :cin] = \
        xin.reshape(H, W, cin).astype(jnp.bfloat16)
    col3 = _row_slab(pad_ref, cin, H, W)
    return _row_dots(col3, w_ref, b_ref, cin, H, W)


def _conv_from_input(f_ref, w_ref, b_ref, H, W):
    cin = w_ref.shape[0] // 9
    col3 = _row_slab(f_ref, cin, H, W)
    return _row_dots(col3, w_ref, b_ref, cin, H, W)


def _pred_diff(m, wd_ref, bd_ref, H, W):
    hw = H * W
    q = jnp.dot(jnp.maximum(m, 0.0).astype(jnp.bfloat16), wd_ref[...],
                preferred_element_type=jnp.float32)
    qp = jnp.pad(q, ((W + 1, W + 1), (0, 0)))
    wcol = jax.lax.broadcasted_iota(jnp.int32, (hw, 1), 0) % W
    d = jnp.zeros((hw, 1), jnp.float32)
    for dy in range(3):
        for dx in range(3):
            k = dy * 3 + dx
            s = (dy - 1) * W + (dx - 1)
            col = qp[W + 1 + s:W + 1 + s + hw, k:k + 1]
            if dx == 0:
                col = jnp.where(wcol == 0, 0.0, col)
            elif dx == 2:
                col = jnp.where(wcol == W - 1, 0.0, col)
            d = d + col
    return d + bd_ref[...]


def _zero_border(pad_ref, H, W):
    c = pad_ref.shape[-1]
    wp = pad_ref.shape[1]
    z_row = jnp.zeros((1, wp, c), jnp.bfloat16)
    z_col = jnp.zeros((H + 2, 1, c), jnp.bfloat16)
    pad_ref[0:1, :, :] = z_row
    pad_ref[H + 1:H + 2, :, :] = z_row
    pad_ref[:, _PADL - 1:_PADL, :] = z_col
    pad_ref[:, _PADL + W:_PADL + W + 1, :] = z_col


def _pipe_kernel(f3_ref, f2_ref, f1_ref, kr2_ref, kr1_ref,
                 w0, b0, w1, b1, w2, b2,
                 fs2, bfs2, a2, ba2, c2, bc2, d2, bd2, e2, be2,
                 fs1, bfs1, a1, ba1, c1, bc1, d1, bd1, e1, be1,
                 wd, bdd, out_ref, pad3, pad2, pad1,
                 *, H3, W3, H2, W2, H1, W1):
    _zero_border(pad3, H3, W3)
    _zero_border(pad2, H2, W2)
    _zero_border(pad1, H1, W1)

    x = _conv_from_input(f3_ref, w0, b0, H3, W3)
    r = _conv(x, pad3, w1, b1, H3, W3, relu_in=True)
    r = _conv(r, pad3, w2, b2, H3, W3, relu_in=True)
    m3 = (x + r).astype(jnp.bfloat16)

    up = jnp.dot(kr2_ref[...], m3, preferred_element_type=jnp.float32)
    x = _conv_from_input(f2_ref, fs2, bfs2, H2, W2)
    r = _conv(x, pad2, a2, ba2, H2, W2, relu_in=True)
    r = _conv(r, pad2, c2, bc2, H2, W2, relu_in=True)
    m = x + r + up
    r = _conv(m, pad2, d2, bd2, H2, W2, relu_in=True)
    m2 = (m + _conv(r, pad2, e2, be2, H2, W2, relu_in=True)).astype(jnp.bfloat16)

    up = jnp.dot(kr1_ref[...], m2, preferred_element_type=jnp.float32)
    x = _conv_from_input(f1_ref, fs1, bfs1, H1, W1)
    r = _conv(x, pad1, a1, ba1, H1, W1, relu_in=True)
    r = _conv(r, pad1, c1, bc1, H1, W1, relu_in=True)
    m = x + r + up
    r = _conv(m, pad1, d1, bd1, H1, W1, relu_in=True)
    m = m + _conv(r, pad1, e1, be1, H1, W1, relu_in=True)
    d = _conv(m, pad1, wd, bdd, H1, W1, relu_in=True)
    out_ref[...] = d.reshape(1, H1 * W1)


def _pipeline(f3, f2, f1, kr2, kr1, weights, dims):
    H3, W3, H2, W2, H1, W1 = dims
    N = f3.shape[0]
    args = [f3, f2, f1, kr2, kr1] + weights
    in_specs = [pl.BlockSpec((None,) + f3.shape[1:], lambda n: (n, 0, 0, 0)),
                pl.BlockSpec((None,) + f2.shape[1:], lambda n: (n, 0, 0, 0)),
                pl.BlockSpec((None,) + f1.shape[1:], lambda n: (n, 0, 0, 0))]
    in_specs += [pl.BlockSpec(a.shape, lambda n: (0, 0)) for a in args[3:]]
    kern = functools.partial(_pipe_kernel, H3=H3, W3=W3, H2=H2, W2=W2,
                             H1=H1, W1=W1)
    return pl.pallas_call(
        kern,
        out_shape=jax.ShapeDtypeStruct((N, 1, H1 * W1), jnp.float32),
        grid=(N,),
        in_specs=in_specs,
        out_specs=pl.BlockSpec((None, 1, H1 * W1), lambda n: (n, 0, 0)),
        scratch_shapes=[
            pltpu.VMEM((H3 + 2, W3 + _PADW, 128), jnp.bfloat16),
            pltpu.VMEM((H2 + 2, W2 + _PADW, 128), jnp.bfloat16),
            pltpu.VMEM((H1 + 2, W1 + _PADW, 128), jnp.bfloat16),
        ],
        compiler_params=pltpu.CompilerParams(dimension_semantics=("arbitrary",)),
    )(*args)


def _agg_kernel(d_ref, a4_ref, bt_ref, out_ref, *, no, max_obj, hf, wf):
    eps = 1e-7

    def logit(e):
        e = jnp.clip(e, eps, 1.0 - eps)
        return jnp.log(e / (1.0 - e))

    t = jnp.dot(a4_ref[...], d_ref[...], preferred_element_type=jnp.float32)
    u = jnp.dot(t, bt_ref[...], preferred_element_type=jnp.float32)
    ps = jax.nn.sigmoid(u)
    rows = [ps[i * hf:(i + 1) * hf] for i in range(no)]
    bg = 1.0 - rows[0]
    for i in range(1, no):
        bg = bg * (1.0 - rows[i])
    out_ref[0:1] = logit(bg)[None]
    for i in range(no):
        out_ref[1 + i:2 + i] = logit(rows[i])[None]
    pad_val = math.log(eps / (1.0 - eps))
    for j in range(no + 1, max_obj):
        out_ref[j:j + 1] = jnp.full((1, hf, wf), pad_val, jnp.float32)


def _aggregate(d3, a4, bt, no, max_obj, hf, wf):
    bs = d3.shape[0]
    args = [d3, a4, bt]
    in_specs = [pl.BlockSpec((None,) + d3.shape[1:], lambda n: (n, 0, 0)),
                pl.BlockSpec(a4.shape, lambda n: (0, 0)),
                pl.BlockSpec(bt.shape, lambda n: (0, 0))]
    return pl.pallas_call(
        functools.partial(_agg_kernel, no=no, max_obj=max_obj, hf=hf, wf=wf),
        out_shape=jax.ShapeDtypeStruct((bs, max_obj, hf, wf), jnp.float32),
        grid=(bs,),
        in_specs=in_specs,
        out_specs=pl.BlockSpec((None, max_obj, hf, wf), lambda n: (n, 0, 0, 0)),
        compiler_params=pltpu.CompilerParams(dimension_semantics=("arbitrary",)),
    )(*args)


def _to_padded_windows(x):
    N, C, H, W = x.shape
    t = jnp.transpose(x.reshape(N, C, H * W), (0, 2, 1)).astype(jnp.bfloat16)
    t = t.reshape(N, H, W, C)
    return jnp.pad(t, ((0, 0), (1, 1), (_PADL, _PADW - _PADL), (0, 0)))


def kernel(convFM_w, convFM_b, resMM_w1, resMM_b1, resMM_w2, resMM_b2,
           rf2_convFS_w, rf2_convFS_b, rf2_resFS_w1, rf2_resFS_b1,
           rf2_resFS_w2, rf2_resFS_b2, rf2_resMM_w1, rf2_resMM_b1,
           rf2_resMM_w2, rf2_resMM_b2, rf1_convFS_w, rf1_convFS_b,
           rf1_resFS_w1, rf1_resFS_b1, rf1_resFS_w2, rf1_resFS_b2,
           rf1_resMM_w1, rf1_resMM_b1, rf1_resMM_w2, rf1_resMM_b2,
           pred_w, pred_b, opt_feat, r2, r1):
    bs, max_obj, h, w = 4, 5, 256, 256
    N, c_in, H3, W3 = opt_feat.shape
    _, c_in_2, H2, W2 = r2.shape
    _, c_in_1, H1, W1 = r1.shape
    no = N // bs

    bf16 = jnp.bfloat16
    cast = lambda a: a.astype(bf16)

    f3 = _to_padded_windows(opt_feat)
    f2 = _to_padded_windows(r2)
    f1 = _to_padded_windows(r1)

    kr2 = jnp.asarray(_upsample_kron_np(H2, W2, H3, W3), bf16)
    kr1 = jnp.asarray(_upsample_kron_np(H1, W1, H2, W2), bf16)

    wd = cast(pred_w[:, 1:2] - pred_w[:, 0:1])
    bd = (pred_b[:, 1:2] - pred_b[:, 0:1]).astype(jnp.float32)

    weights = [kr2, kr1,
               cast(convFM_w), convFM_b, cast(resMM_w1), resMM_b1,
               cast(resMM_w2), resMM_b2,
               cast(rf2_convFS_w), rf2_convFS_b,
               cast(rf2_resFS_w1), rf2_resFS_b1,
               cast(rf2_resFS_w2), rf2_resFS_b2,
               cast(rf2_resMM_w1), rf2_resMM_b1,
               cast(rf2_resMM_w2), rf2_resMM_b2,
               cast(rf1_convFS_w), rf1_convFS_b,
               cast(rf1_resFS_w1), rf1_resFS_b1,
               cast(rf1_resFS_w2), rf1_resFS_b2,
               cast(rf1_resMM_w1), rf1_resMM_b1,
               cast(rf1_resMM_w2), rf1_resMM_b2,
               wd, bd]

    d = _pipeline(f3, f2, f1, weights[0], weights[1], weights[2:],
                  (H3, W3, H2, W2, H1, W1))

    d3 = d.reshape(bs, no * H1, W1)
    a4 = jnp.asarray(np.kron(np.eye(no, dtype=np.float32),
                             _interp_matrix_np(h, H1)), jnp.float32)
    bt = jnp.asarray(_interp_matrix_np(w, W1).T, jnp.float32)
    return _aggregate(d3, a4, bt, no, max_obj, h, w)
